# baseline (speedup 1.0000x reference)
import jax
import jax.numpy as jnp
from jax.experimental import pallas as pl
from jax.experimental.pallas import tpu as pltpu

_BM = 256
_NBUF = 12


def _spmm_body(adj_hbm, emb_ref, out_ref, bufs, sems):
    nchunk = adj_hbm.shape[0] // _BM

    def _copy(i):
        return pltpu.make_async_copy(
            adj_hbm.at[pl.ds(i * _BM, _BM), :],
            bufs.at[i % _NBUF],
            sems.at[i % _NBUF],
        )

    for i in range(min(_NBUF, nchunk)):
        _copy(i).start()
    for i in range(nchunk):
        _copy(i).wait()
        out_ref[pl.ds(i * _BM, _BM), :] = jnp.dot(
            bufs[i % _NBUF], emb_ref[...], preferred_element_type=jnp.float32
        )
        if i + _NBUF < nchunk:
            _copy(i + _NBUF).start()


def kernel(adj, embeds):
    M, K = adj.shape
    _, N = embeds.shape
    return pl.pallas_call(
        _spmm_body,
        in_specs=[
            pl.BlockSpec(memory_space=pltpu.MemorySpace.HBM),
            pl.BlockSpec((K, N), lambda: (0, 0)),
        ],
        out_specs=pl.BlockSpec((M, N), lambda: (0, 0)),
        out_shape=jax.ShapeDtypeStruct((M, N), jnp.float32),
        scratch_shapes=[
            pltpu.VMEM((_NBUF, _BM, K), jnp.float32),
            pltpu.SemaphoreType.DMA((_NBUF,)),
        ],
    )(adj, embeds)
